# baseline (speedup 1.0000x reference)
import functools

import jax
import jax.numpy as jnp
from jax.experimental import pallas as pl
from jax.experimental.pallas import tpu as pltpu


def _matmul_kernel(x_ref, w_ref, b_ref, o_ref, *, tail):
    i = pl.program_id(0)
    last = pl.num_programs(0) - 1

    @pl.when(i == 0)
    def _init():
        o_ref[...] = jnp.broadcast_to(b_ref[...], o_ref.shape)

    if tail is None:
        o_ref[...] += jnp.dot(
            x_ref[...], w_ref[...], preferred_element_type=jnp.float32
        )
    else:
        @pl.when(i != last)
        def _body():
            o_ref[...] += jnp.dot(
                x_ref[...], w_ref[...], preferred_element_type=jnp.float32
            )

        @pl.when(i == last)
        def _tail():
            x = x_ref[...]
            w = w_ref[...]
            col = jax.lax.broadcasted_iota(jnp.int32, x.shape, 1)
            row = jax.lax.broadcasted_iota(jnp.int32, w.shape, 0)
            xm = jnp.where(col < tail, x, 0.0)
            wm = jnp.where(row < tail, w, 0.0)
            o_ref[...] += jnp.dot(xm, wm, preferred_element_type=jnp.float32)


@functools.partial(jax.jit, static_argnames=())
def kernel(x, kernel, bias):
    b, v = x.shape
    n = kernel.shape[1]
    kv = 4096
    steps = -(-v // kv)
    rem = v - (steps - 1) * kv
    tail = None if rem == kv else rem
    bias2 = bias.reshape(1, n)
    out = pl.pallas_call(
        functools.partial(_matmul_kernel, tail=tail),
        grid=(steps,),
        in_specs=[
            pl.BlockSpec((b, kv), lambda i: (0, i)),
            pl.BlockSpec((kv, n), lambda i: (i, 0)),
            pl.BlockSpec((1, n), lambda i: (0, 0)),
        ],
        out_specs=pl.BlockSpec((b, n), lambda i: (0, 0)),
        out_shape=jax.ShapeDtypeStruct((b, n), jnp.float32),
        compiler_params=pltpu.CompilerParams(
            dimension_semantics=("arbitrary",),
        ),
    )(x, kernel, bias2)
    return out
